# Initial kernel scaffold; baseline (speedup 1.0000x reference)
import jax
import jax.numpy as jnp
from jax import lax
from jax.experimental import pallas as pl
from jax.experimental.pallas import tpu as pltpu
from jax.experimental.pallas import tpu_sc as plsc

NC = 2
NS = 16
NW = NC * NS
K = 128
F32 = jnp.float32


def _mesh():
    return plsc.VectorSubcoreMesh(core_axis_name="c", subcore_axis_name="s",
                                  num_cores=NC, num_subcores=NS)


def _cdiv(a, b):
    return -(-a // b)


def _round_up(a, b):
    return _cdiv(a, b) * b


def _sc_degree(idx, num_rows):
    cd = idx.shape[2]
    rps = num_rows // NS

    def body(idx_hbm, out_hbm, zeros_v, ones_v, idx_v, deg_sh):
        cid = lax.axis_index("c")
        sid = lax.axis_index("s")

        @pl.loop(0, 128)
        def _(i):
            zeros_v[i, pl.ds(0, 16)] = jnp.zeros((16,), F32)
            ones_v[i, pl.ds(0, 16)] = jnp.ones((16,), F32)

        base = sid * rps

        @pl.loop(0, rps // 128)
        def _(t):
            pltpu.sync_copy(zeros_v, deg_sh.at[pl.ds(base + t * 128, 128)])

        plsc.subcore_barrier()

        @pl.loop(0, cd)
        def _(c):
            pltpu.sync_copy(idx_hbm.at[cid, sid, c], idx_v)
            pltpu.sync_copy(ones_v, deg_sh.at[idx_v], add=True)

        plsc.subcore_barrier()
        pltpu.sync_copy(deg_sh.at[pl.ds(base, rps)],
                        out_hbm.at[cid, pl.ds(base, rps)])

    return pl.kernel(
        body,
        out_type=jax.ShapeDtypeStruct((NC, num_rows, 16), F32),
        mesh=_mesh(),
        scratch_types=[
            pltpu.VMEM((128, 16), F32),
            pltpu.VMEM((128, 16), F32),
            pltpu.VMEM((K,), jnp.int32),
            pltpu.VMEM_SHARED((num_rows, 16), F32),
        ],
    )(idx)


def _sc_prop(z, src_idx, dst_idx, n_pad):
    c_chunks = src_idx.shape[2]
    d = z.shape[1]
    rps = n_pad // NS

    def body(z_hbm, src_hbm, dst_hbm, out_hbm,
             zeros_v, src_v, dst_v, rows_v, acc_sh, sem):
        cid = lax.axis_index("c")
        sid = lax.axis_index("s")

        @pl.loop(0, 64)
        def _(i):
            @pl.loop(0, d // 16)
            def _(j):
                zeros_v[i, pl.ds(j * 16, 16)] = jnp.zeros((16,), F32)

        base = sid * rps

        @pl.loop(0, rps // 64)
        def _(t):
            pltpu.sync_copy(zeros_v, acc_sh.at[pl.ds(base + t * 64, 64)])

        plsc.subcore_barrier()

        @pl.loop(0, c_chunks)
        def _(c):
            pltpu.sync_copy(src_hbm.at[cid, sid, c], src_v)
            pltpu.sync_copy(dst_hbm.at[cid, sid, c], dst_v)
            pltpu.async_copy(z_hbm.at[src_v], rows_v, sem).wait()
            pltpu.sync_copy(rows_v, acc_sh.at[dst_v], add=True)

        plsc.subcore_barrier()
        pltpu.sync_copy(acc_sh.at[pl.ds(base, rps)],
                        out_hbm.at[cid, pl.ds(base, rps)])

    return pl.kernel(
        body,
        out_type=jax.ShapeDtypeStruct((NC, n_pad, d), F32),
        mesh=_mesh(),
        scratch_types=[
            pltpu.VMEM((64, d), F32),
            pltpu.VMEM((K,), jnp.int32),
            pltpu.VMEM((K,), jnp.int32),
            pltpu.VMEM((K, d), F32),
            pltpu.VMEM_SHARED((n_pad, d), F32),
            pltpu.SemaphoreType.DMA,
        ],
    )(z, src_idx, dst_idx)


def _bcast_lanes(v16, d):
    return jnp.concatenate([v16] * (d // 16), axis=1)


def _tc_pre(x, dp, n):
    d = x.shape[1]

    def body(x_ref, dp_ref, z_ref):
        deg = dp_ref[0, pl.ds(0, n), :] + dp_ref[1, pl.ds(0, n), :]
        nsrc = lax.rsqrt(jnp.maximum(deg, 1.0))
        z_ref[...] = x_ref[...] * _bcast_lanes(nsrc, d)

    return pl.pallas_call(
        body, out_shape=jax.ShapeDtypeStruct((n, d), F32))(x, dp)


def _tc_layer(acc, dp, w, b, n, scale_src):
    d = w.shape[0]

    def body(acc_ref, dp_ref, w_ref, b_ref, o_ref):
        s = acc_ref[0, pl.ds(0, n), :] + acc_ref[1, pl.ds(0, n), :]
        dd = dp_ref[0, pl.ds(n, n), :] + dp_ref[1, pl.ds(n, n), :]
        ndst = lax.rsqrt(jnp.maximum(dd, 1.0))
        if scale_src:
            ds_ = dp_ref[0, pl.ds(0, n), :] + dp_ref[1, pl.ds(0, n), :]
            nsrc = lax.rsqrt(jnp.maximum(ds_, 1.0))
            scale = ndst * nsrc
        else:
            scale = ndst
        sm = s * _bcast_lanes(scale, d)
        out = jnp.dot(sm, w_ref[...], preferred_element_type=F32,
                      precision=lax.Precision.HIGHEST)
        if scale_src:
            out = out + _bcast_lanes(nsrc, d) * b_ref[...]
        else:
            out = out + b_ref[...]
        o_ref[...] = out

    return pl.pallas_call(
        body, out_shape=jax.ShapeDtypeStruct((n, d), F32))(acc, dp, w, b)


def kernel(in_feat, edge_index, W1, b1, W2, b2):
    n, d = in_feat.shape
    e = edge_index.shape[1]
    src = edge_index[0]
    dst = edge_index[1]

    c_chunks = _cdiv(_cdiv(e, NW), K)
    pad = NW * c_chunks * K - e
    srcp = jnp.concatenate(
        [src, jnp.zeros((pad,), jnp.int32)]).reshape(NC, NS, c_chunks, K)
    dstp = jnp.concatenate(
        [dst, jnp.full((pad,), n, jnp.int32)]).reshape(NC, NS, c_chunks, K)

    cd_chunks = _cdiv(_cdiv(2 * e, NW), K)
    dpad = NW * cd_chunks * K - 2 * e
    didx = jnp.concatenate(
        [src, dst + n, jnp.full((dpad,), 2 * n, jnp.int32)]
    ).reshape(NC, NS, cd_chunks, K)

    nd_rows = _round_up(2 * n + 1, NS * 128)
    n_pad = _round_up(n + 1, NS * 64)

    dp = _sc_degree(didx, nd_rows)
    z = _tc_pre(in_feat, dp, n)
    acc1 = _sc_prop(z, srcp, dstp, n_pad)
    z2 = _tc_layer(acc1, dp, W1, b1.reshape(1, d), n, scale_src=True)
    acc2 = _sc_prop(z2, srcp, dstp, n_pad)
    out = _tc_layer(acc2, dp, W2, b2.reshape(1, d), n, scale_src=False)
    return out


# speedup vs baseline: 3.7722x; 3.7722x over previous
import jax
import jax.numpy as jnp
from jax import lax
from jax.experimental import pallas as pl
from jax.experimental.pallas import tpu as pltpu
from jax.experimental.pallas import tpu_sc as plsc

NC = 2
NS = 16
NW = NC * NS
K = 128
F32 = jnp.float32


def _mesh():
    return plsc.VectorSubcoreMesh(core_axis_name="c", subcore_axis_name="s",
                                  num_cores=NC, num_subcores=NS)


def _cdiv(a, b):
    return -(-a // b)


def _round_up(a, b):
    return _cdiv(a, b) * b


def _sc_degree(src_idx, dst_idx, n_pad, d):
    c_chunks = src_idx.shape[2]
    rps = n_pad // NS

    def body(src_hbm, dst_hbm, out_hbm,
             zeros_v, ones_s, ones_d, src_v, dst_v, acc_sh):
        cid = lax.axis_index("c")
        sid = lax.axis_index("s")

        @pl.loop(0, 64)
        def _(i):
            @pl.loop(0, d // 16)
            def _(j):
                zeros_v[i, pl.ds(j * 16, 16)] = jnp.zeros((16,), F32)

        @pl.loop(0, K)
        def _(i):
            @pl.loop(0, d // 16)
            def _(j):
                ones_s[i, pl.ds(j * 16, 16)] = jnp.zeros((16,), F32)
                ones_d[i, pl.ds(j * 16, 16)] = jnp.zeros((16,), F32)
            ones_s[i, pl.ds(0, 16)] = jnp.ones((16,), F32)
            ones_d[i, pl.ds(16, 16)] = jnp.ones((16,), F32)

        base = sid * rps

        @pl.loop(0, rps // 64)
        def _(t):
            pltpu.sync_copy(zeros_v, acc_sh.at[pl.ds(base + t * 64, 64)])

        plsc.subcore_barrier()

        @pl.loop(0, c_chunks)
        def _(c):
            pltpu.sync_copy(src_hbm.at[cid, sid, c], src_v)
            pltpu.sync_copy(dst_hbm.at[cid, sid, c], dst_v)
            pltpu.sync_copy(ones_s, acc_sh.at[src_v], add=True)
            pltpu.sync_copy(ones_d, acc_sh.at[dst_v], add=True)

        plsc.subcore_barrier()
        pltpu.sync_copy(acc_sh.at[pl.ds(base, rps)],
                        out_hbm.at[cid, pl.ds(base, rps)])

    return pl.kernel(
        body,
        out_type=jax.ShapeDtypeStruct((NC, n_pad, d), F32),
        mesh=_mesh(),
        scratch_types=[
            pltpu.VMEM((64, d), F32),
            pltpu.VMEM((K, d), F32),
            pltpu.VMEM((K, d), F32),
            pltpu.VMEM((K,), jnp.int32),
            pltpu.VMEM((K,), jnp.int32),
            pltpu.VMEM_SHARED((n_pad, d), F32),
        ],
    )(src_idx, dst_idx)


def _sc_prop(z, src_idx, dst_idx, n_pad):
    c_chunks = src_idx.shape[2]
    d = z.shape[1]
    rps = n_pad // NS

    def body(z_hbm, src_hbm, dst_hbm, out_hbm,
             zeros_v, src_v, dst_v, rows_v, acc_sh, sem):
        cid = lax.axis_index("c")
        sid = lax.axis_index("s")

        @pl.loop(0, 64)
        def _(i):
            @pl.loop(0, d // 16)
            def _(j):
                zeros_v[i, pl.ds(j * 16, 16)] = jnp.zeros((16,), F32)

        base = sid * rps

        @pl.loop(0, rps // 64)
        def _(t):
            pltpu.sync_copy(zeros_v, acc_sh.at[pl.ds(base + t * 64, 64)])

        plsc.subcore_barrier()

        @pl.loop(0, c_chunks)
        def _(c):
            pltpu.sync_copy(src_hbm.at[cid, sid, c], src_v)
            pltpu.sync_copy(dst_hbm.at[cid, sid, c], dst_v)
            pltpu.async_copy(z_hbm.at[src_v], rows_v, sem).wait()
            pltpu.sync_copy(rows_v, acc_sh.at[dst_v], add=True)

        plsc.subcore_barrier()
        pltpu.sync_copy(acc_sh.at[pl.ds(base, rps)],
                        out_hbm.at[cid, pl.ds(base, rps)])

    return pl.kernel(
        body,
        out_type=jax.ShapeDtypeStruct((NC, n_pad, d), F32),
        mesh=_mesh(),
        scratch_types=[
            pltpu.VMEM((64, d), F32),
            pltpu.VMEM((K,), jnp.int32),
            pltpu.VMEM((K,), jnp.int32),
            pltpu.VMEM((K, d), F32),
            pltpu.VMEM_SHARED((n_pad, d), F32),
            pltpu.SemaphoreType.DMA,
        ],
    )(z, src_idx, dst_idx)


def _bcast_lanes(v16, d):
    return jnp.concatenate([v16] * (d // 16), axis=1)


def _row_block(n):
    for br in (2000, 1000, 500, 200, 100):
        if n % br == 0 and br % 8 == 0:
            return br
    return n


def _tc_pre(x, dps, n):
    d = x.shape[1]
    br = _row_block(n)

    def body(x_ref, dp_ref, z_ref):
        deg = dp_ref[0] + dp_ref[1]
        nsrc = lax.rsqrt(jnp.maximum(deg, 1.0))
        z_ref[...] = x_ref[...] * _bcast_lanes(nsrc, d)

    return pl.pallas_call(
        body,
        grid=(n // br,),
        in_specs=[
            pl.BlockSpec((br, d), lambda i: (i, 0)),
            pl.BlockSpec((NC, br, 16), lambda i: (0, i, 0)),
        ],
        out_specs=pl.BlockSpec((br, d), lambda i: (i, 0)),
        out_shape=jax.ShapeDtypeStruct((n, d), F32),
    )(x, dps)


def _tc_layer(acc, dps, dpd, w, b, n, scale_src):
    d = w.shape[0]
    br = _row_block(n)

    def body(acc_ref, dps_ref, dpd_ref, w_ref, b_ref, o_ref):
        s = acc_ref[0] + acc_ref[1]
        ndst = lax.rsqrt(jnp.maximum(dpd_ref[0] + dpd_ref[1], 1.0))
        if scale_src:
            nsrc = lax.rsqrt(jnp.maximum(dps_ref[0] + dps_ref[1], 1.0))
            scale = ndst * nsrc
        else:
            scale = ndst
        sm = s * _bcast_lanes(scale, d)
        out = jnp.dot(sm, w_ref[...], preferred_element_type=F32,
                      precision=lax.Precision.HIGHEST)
        if scale_src:
            out = out + _bcast_lanes(nsrc, d) * b_ref[...]
        else:
            out = out + b_ref[...]
        o_ref[...] = out

    return pl.pallas_call(
        body,
        grid=(n // br,),
        in_specs=[
            pl.BlockSpec((NC, br, d), lambda i: (0, i, 0)),
            pl.BlockSpec((NC, br, 16), lambda i: (0, i, 0)),
            pl.BlockSpec((NC, br, 16), lambda i: (0, i, 0)),
            pl.BlockSpec((d, d), lambda i: (0, 0)),
            pl.BlockSpec((1, d), lambda i: (0, 0)),
        ],
        out_specs=pl.BlockSpec((br, d), lambda i: (i, 0)),
        out_shape=jax.ShapeDtypeStruct((n, d), F32),
    )(acc, dps, dpd, w, b)


def kernel(in_feat, edge_index, W1, b1, W2, b2):
    n, d = in_feat.shape
    e = edge_index.shape[1]
    src = edge_index[0]
    dst = edge_index[1]

    c_chunks = _cdiv(_cdiv(e, NW), K)
    pad = NW * c_chunks * K - e
    srcp = jnp.concatenate(
        [src, jnp.full((pad,), n, jnp.int32)]).reshape(NC, NS, c_chunks, K)
    dstp = jnp.concatenate(
        [dst, jnp.full((pad,), n, jnp.int32)]).reshape(NC, NS, c_chunks, K)

    n_pad = _round_up(n + 1, NS * 64)
    zpad = jnp.zeros((8, d), F32)

    dp = _sc_degree(srcp, dstp, n_pad, d)
    dps = dp[:, :n, 0:16]
    dpd = dp[:, :n, 16:32]
    z = _tc_pre(in_feat, dps, n)
    acc1 = _sc_prop(jnp.concatenate([z, zpad]), srcp, dstp, n_pad)
    z2 = _tc_layer(acc1[:, :n, :], dps, dpd, W1, b1.reshape(1, d), n,
                   scale_src=True)
    acc2 = _sc_prop(jnp.concatenate([z2, zpad]), srcp, dstp, n_pad)
    out = _tc_layer(acc2[:, :n, :], dps, dpd, W2, b2.reshape(1, d), n,
                    scale_src=False)
    return out
